# baseline (speedup 1.0000x reference)
import jax
import jax.numpy as jnp
from jax.experimental import pallas as pl
from jax.experimental.pallas import tpu as pltpu

NCLS = 21
THR = 0.5
NEGPOS = 3
PP = 25500
RR = 200
LL = 128
PPAD = RR * LL
NT = 10
NOUT = 8


def _smooth_l1(d):
    ad = jnp.abs(d)
    return jnp.where(ad < 1.0, 0.5 * d * d, ad - 0.5)


def _loss_body(tgt_ref, pri_ref, loc_ref, conf_ref, obj_ref, out_ref):
    b = pl.program_id(0)

    pcx = pri_ref[0]
    pcy = pri_ref[1]
    pw = pri_ref[2]
    ph = pri_ref[3]
    px1 = pcx - pw * 0.5
    py1 = pcy - ph * 0.5
    px2 = pcx + pw * 0.5
    py2 = pcy + ph * 0.5
    parea = (px2 - px1) * (py2 - py1)

    row = jax.lax.broadcasted_iota(jnp.int32, (RR, LL), 0)
    col = jax.lax.broadcasted_iota(jnp.int32, (RR, LL), 1)
    p_idx = row * LL + col
    valid = p_idx < PP

    bto = jnp.full((RR, LL), -1.0, jnp.float32)
    bti = jnp.zeros((RR, LL), jnp.int32)
    tx1 = [tgt_ref[b, 5 * j + 0] for j in range(NT)]
    ty1 = [tgt_ref[b, 5 * j + 1] for j in range(NT)]
    tx2 = [tgt_ref[b, 5 * j + 2] for j in range(NT)]
    ty2 = [tgt_ref[b, 5 * j + 3] for j in range(NT)]
    tlab = [tgt_ref[b, 5 * j + 4] for j in range(NT)]
    bpi = []
    for j in range(NT):
        ix = jnp.maximum(jnp.minimum(px2, tx2[j]) - jnp.maximum(px1, tx1[j]), 0.0)
        iy = jnp.maximum(jnp.minimum(py2, ty2[j]) - jnp.maximum(py1, ty1[j]), 0.0)
        inter = ix * iy
        tarea = (tx2[j] - tx1[j]) * (ty2[j] - ty1[j])
        ov = inter / (tarea + parea - inter)
        ov = jnp.where(valid, ov, -1.0)
        upd = ov > bto
        bto = jnp.where(upd, ov, bto)
        bti = jnp.where(upd, j, bti)
        mj = jnp.max(ov)
        bpi.append(jnp.min(jnp.where(ov == mj, p_idx, PPAD)))
    for j in range(NT):
        hit = p_idx == bpi[j]
        bto = jnp.where(hit, 2.0, bto)
        bti = jnp.where(hit, j, bti)
    mx1 = jnp.zeros((RR, LL), jnp.float32)
    my1 = jnp.zeros((RR, LL), jnp.float32)
    mx2 = jnp.zeros((RR, LL), jnp.float32)
    my2 = jnp.zeros((RR, LL), jnp.float32)
    lab = jnp.zeros((RR, LL), jnp.float32)
    for j in range(NT):
        sel = bti == j
        mx1 = jnp.where(sel, tx1[j], mx1)
        my1 = jnp.where(sel, ty1[j], my1)
        mx2 = jnp.where(sel, tx2[j], mx2)
        my2 = jnp.where(sel, ty2[j], my2)
        lab = jnp.where(sel, tlab[j], lab)
    conf_t = jnp.where(bto < THR, 0, (lab + 1.0).astype(jnp.int32))
    pos = conf_t > 0

    g0 = ((mx1 + mx2) * 0.5 - pcx) / (0.1 * pw)
    g1 = ((my1 + my2) * 0.5 - pcy) / (0.1 * ph)
    g2 = jnp.log((mx2 - mx1) / pw) / 0.2
    g3 = jnp.log((my2 - my1) / ph) / 0.2
    sl1 = (_smooth_l1(loc_ref[0, 0] - g0) + _smooth_l1(loc_ref[0, 1] - g1)
           + _smooth_l1(loc_ref[0, 2] - g2) + _smooth_l1(loc_ref[0, 3] - g3))
    loss_l = jnp.sum(jnp.where(pos, sl1, 0.0))

    o0 = obj_ref[0, 0]
    o1 = obj_ref[0, 1]
    mo = jnp.maximum(o0, o1)
    lse_o = jnp.log(jnp.exp(o0 - mo) + jnp.exp(o1 - mo)) + mo
    ce_o = lse_o - jnp.where(pos, o1, o0)

    mc = conf_ref[0, 0]
    for i in range(1, NCLS):
        mc = jnp.maximum(mc, conf_ref[0, i])
    s = jnp.zeros((RR, LL), jnp.float32)
    chosen = jnp.zeros((RR, LL), jnp.float32)
    for i in range(NCLS):
        ci = conf_ref[0, i]
        s = s + jnp.exp(ci - mc)
        chosen = jnp.where(conf_t == i, ci, chosen)
    ce_c = (jnp.log(s) + mc) - chosen

    mine_c = jnp.where(valid, jnp.where(pos, 0.0, ce_c), -1.0)
    mine_o = jnp.where(valid, jnp.where(pos, 0.0, ce_o), -1.0)
    vi_c = jax.lax.bitcast_convert_type(mine_c, jnp.int32)
    vi_o = jax.lax.bitcast_convert_type(mine_o, jnp.int32)
    np_cnt = jnp.sum(pos.astype(jnp.int32))
    k = jnp.minimum(NEGPOS * np_cnt, PP - 1)

    def bis(i, st):
        lo_c, hi_c, lo_o, hi_o = st
        mid_c = lo_c + (hi_c - lo_c + 1) // 2
        ok_c = jnp.sum((vi_c >= mid_c).astype(jnp.int32)) >= k
        mid_o = lo_o + (hi_o - lo_o + 1) // 2
        ok_o = jnp.sum((vi_o >= mid_o).astype(jnp.int32)) >= k
        return (jnp.where(ok_c, mid_c, lo_c), jnp.where(ok_c, hi_c, mid_c - 1),
                jnp.where(ok_o, mid_o, lo_o), jnp.where(ok_o, hi_o, mid_o - 1))

    t_c, _, t_o, _ = jax.lax.fori_loop(
        0, 31, bis,
        (jnp.int32(0), jnp.max(vi_c), jnp.int32(0), jnp.max(vi_o)))

    def neg_sum(mine, t_int):
        t = jax.lax.bitcast_convert_type(t_int, jnp.float32)
        gt = mine > t
        g_cnt = jnp.sum(gt.astype(jnp.int32))
        return (jnp.sum(jnp.where(gt, mine, 0.0))
                + (k - g_cnt).astype(jnp.float32) * t)

    loss_c = jnp.sum(jnp.where(pos, ce_c, 0.0)) + neg_sum(mine_c, t_c)
    loss_o = jnp.sum(jnp.where(pos, ce_o, 0.0)) + neg_sum(mine_o, t_o)

    out_ref[0, 0, 0] = loss_l
    out_ref[0, 0, 1] = loss_c
    out_ref[0, 0, 2] = loss_o
    out_ref[0, 0, 3] = np_cnt.astype(jnp.float32)
    out_ref[0, 0, 4] = k.astype(jnp.float32)
    out_ref[0, 0, 5] = 0.0
    out_ref[0, 0, 6] = 0.0
    out_ref[0, 0, 7] = 0.0


CHUNK = 8


def kernel(loc_data, conf_data, obj_data, priors, targets):
    bsz = loc_data.shape[0]
    pad = PPAD - PP

    def prep(x, cb):
        x = jnp.pad(x, ((0, 0), (0, pad), (0, 0)))
        return x.transpose(0, 2, 1).reshape(cb, x.shape[2], RR, LL)

    pri_p = jnp.pad(priors, ((0, pad), (0, 0))).T.reshape(4, RR, LL)

    outs = []
    for c0 in range(0, bsz, CHUNK):
        cb = min(CHUNK, bsz - c0)
        loc_p = prep(loc_data[c0:c0 + cb], cb)
        conf_p = prep(conf_data[c0:c0 + cb], cb)
        obj_p = prep(obj_data[c0:c0 + cb], cb)
        tgt = targets[c0:c0 + cb].reshape(cb, NT * 5)
        outs.append(pl.pallas_call(
            _loss_body,
            grid=(cb,),
            in_specs=[
                pl.BlockSpec(memory_space=pltpu.SMEM),
                pl.BlockSpec((4, RR, LL), lambda b: (0, 0, 0)),
                pl.BlockSpec((1, 4, RR, LL), lambda b: (b, 0, 0, 0)),
                pl.BlockSpec((1, NCLS, RR, LL), lambda b: (b, 0, 0, 0)),
                pl.BlockSpec((1, 2, RR, LL), lambda b: (b, 0, 0, 0)),
            ],
            out_specs=pl.BlockSpec((1, 1, NOUT), lambda b: (b, 0, 0),
                                   memory_space=pltpu.SMEM),
            out_shape=jax.ShapeDtypeStruct((cb, 1, NOUT), jnp.float32),
        )(tgt, pri_p, loc_p, conf_p, obj_p))

    o = jnp.concatenate(outs, axis=0).reshape(bsz, NOUT)
    n_pos = jnp.maximum(jnp.sum(o[:, 3]), 1.0)
    n_neg = jnp.maximum(jnp.sum(o[:, 4]), 1.0)
    loss_l = jnp.sum(o[:, 0]) / n_pos
    loss_c = jnp.sum(o[:, 1]) / n_pos
    loss_obj = 0.4 * jnp.sum(o[:, 2]) / n_neg
    return (loss_l, loss_c, loss_obj)


# speedup vs baseline: 1.1321x; 1.1321x over previous
import jax
import jax.numpy as jnp
from jax.experimental import pallas as pl
from jax.experimental.pallas import tpu as pltpu

NCLS = 21
THR = 0.5
NEGPOS = 3
PP = 25500
RR = 200
LL = 128
PPAD = RR * LL
NT = 10
NOUT = 8


def _smooth_l1(d):
    ad = jnp.abs(d)
    return jnp.where(ad < 1.0, 0.5 * d * d, ad - 0.5)


def _loss_body(tgt_ref, pri_ref, loc_ref, conf_ref, obj_ref, out_ref):
    b = pl.program_id(0)

    pcx = pri_ref[0]
    pcy = pri_ref[1]
    pw = pri_ref[2]
    ph = pri_ref[3]
    px1 = pcx - pw * 0.5
    py1 = pcy - ph * 0.5
    px2 = pcx + pw * 0.5
    py2 = pcy + ph * 0.5
    parea = (px2 - px1) * (py2 - py1)

    row = jax.lax.broadcasted_iota(jnp.int32, (RR, LL), 0)
    col = jax.lax.broadcasted_iota(jnp.int32, (RR, LL), 1)
    p_idx = row * LL + col
    valid = p_idx < PP

    bto = jnp.full((RR, LL), -1.0, jnp.float32)
    bti = jnp.zeros((RR, LL), jnp.int32)
    tx1 = [tgt_ref[b, 5 * j + 0] for j in range(NT)]
    ty1 = [tgt_ref[b, 5 * j + 1] for j in range(NT)]
    tx2 = [tgt_ref[b, 5 * j + 2] for j in range(NT)]
    ty2 = [tgt_ref[b, 5 * j + 3] for j in range(NT)]
    tlab = [tgt_ref[b, 5 * j + 4] for j in range(NT)]
    bpi = []
    for j in range(NT):
        ix = jnp.maximum(jnp.minimum(px2, tx2[j]) - jnp.maximum(px1, tx1[j]), 0.0)
        iy = jnp.maximum(jnp.minimum(py2, ty2[j]) - jnp.maximum(py1, ty1[j]), 0.0)
        inter = ix * iy
        tarea = (tx2[j] - tx1[j]) * (ty2[j] - ty1[j])
        ov = inter / (tarea + parea - inter)
        ov = jnp.where(valid, ov, -1.0)
        upd = ov > bto
        bto = jnp.where(upd, ov, bto)
        bti = jnp.where(upd, j, bti)
        mj = jnp.max(ov)
        bpi.append(jnp.min(jnp.where(ov == mj, p_idx, PPAD)))
    for j in range(NT):
        hit = p_idx == bpi[j]
        bto = jnp.where(hit, 2.0, bto)
        bti = jnp.where(hit, j, bti)
    mx1 = jnp.zeros((RR, LL), jnp.float32)
    my1 = jnp.zeros((RR, LL), jnp.float32)
    mx2 = jnp.zeros((RR, LL), jnp.float32)
    my2 = jnp.zeros((RR, LL), jnp.float32)
    lab = jnp.zeros((RR, LL), jnp.float32)
    for j in range(NT):
        sel = bti == j
        mx1 = jnp.where(sel, tx1[j], mx1)
        my1 = jnp.where(sel, ty1[j], my1)
        mx2 = jnp.where(sel, tx2[j], mx2)
        my2 = jnp.where(sel, ty2[j], my2)
        lab = jnp.where(sel, tlab[j], lab)
    conf_t = jnp.where(bto < THR, 0, (lab + 1.0).astype(jnp.int32))
    pos = conf_t > 0

    g0 = ((mx1 + mx2) * 0.5 - pcx) / (0.1 * pw)
    g1 = ((my1 + my2) * 0.5 - pcy) / (0.1 * ph)
    g2 = jnp.log((mx2 - mx1) / pw) / 0.2
    g3 = jnp.log((my2 - my1) / ph) / 0.2
    sl1 = (_smooth_l1(loc_ref[0, 0] - g0) + _smooth_l1(loc_ref[0, 1] - g1)
           + _smooth_l1(loc_ref[0, 2] - g2) + _smooth_l1(loc_ref[0, 3] - g3))
    loss_l = jnp.sum(jnp.where(pos, sl1, 0.0))

    o0 = obj_ref[0, 0]
    o1 = obj_ref[0, 1]
    mo = jnp.maximum(o0, o1)
    lse_o = jnp.log(jnp.exp(o0 - mo) + jnp.exp(o1 - mo)) + mo
    ce_o = lse_o - jnp.where(pos, o1, o0)

    mc = conf_ref[0, 0]
    for i in range(1, NCLS):
        mc = jnp.maximum(mc, conf_ref[0, i])
    s = jnp.zeros((RR, LL), jnp.float32)
    chosen = jnp.zeros((RR, LL), jnp.float32)
    for i in range(NCLS):
        ci = conf_ref[0, i]
        s = s + jnp.exp(ci - mc)
        chosen = jnp.where(conf_t == i, ci, chosen)
    ce_c = (jnp.log(s) + mc) - chosen

    mine_c = jnp.where(valid, jnp.where(pos, 0.0, ce_c), -1.0)
    mine_o = jnp.where(valid, jnp.where(pos, 0.0, ce_o), -1.0)
    vi_c = jax.lax.bitcast_convert_type(mine_c, jnp.int32)
    vi_o = jax.lax.bitcast_convert_type(mine_o, jnp.int32)
    np_cnt = jnp.sum(pos.astype(jnp.int32))
    k = jnp.minimum(NEGPOS * np_cnt, PP - 1)

    def eight_way(lo, hi, vi):
        s = (hi - lo + 8) // 8
        idx = jnp.int32(0)
        for i in range(1, 8):
            cnt = jnp.sum((vi >= lo + i * s).astype(jnp.int32))
            idx = idx + (cnt >= k).astype(jnp.int32)
        new_lo = lo + idx * s
        new_hi = jnp.minimum(hi, lo + (idx + 1) * s - 1)
        return new_lo, new_hi

    def bis(i, st):
        lo_c, hi_c, lo_o, hi_o = st
        lo_c, hi_c = eight_way(lo_c, hi_c, vi_c)
        lo_o, hi_o = eight_way(lo_o, hi_o, vi_o)
        return (lo_c, hi_c, lo_o, hi_o)

    t_c, _, t_o, _ = jax.lax.fori_loop(
        0, 12, bis,
        (jnp.int32(0), jnp.max(vi_c), jnp.int32(0), jnp.max(vi_o)))

    def neg_sum(mine, t_int):
        t = jax.lax.bitcast_convert_type(t_int, jnp.float32)
        gt = mine > t
        g_cnt = jnp.sum(gt.astype(jnp.int32))
        return (jnp.sum(jnp.where(gt, mine, 0.0))
                + (k - g_cnt).astype(jnp.float32) * t)

    loss_c = jnp.sum(jnp.where(pos, ce_c, 0.0)) + neg_sum(mine_c, t_c)
    loss_o = jnp.sum(jnp.where(pos, ce_o, 0.0)) + neg_sum(mine_o, t_o)

    out_ref[0, 0, 0] = loss_l
    out_ref[0, 0, 1] = loss_c
    out_ref[0, 0, 2] = loss_o
    out_ref[0, 0, 3] = np_cnt.astype(jnp.float32)
    out_ref[0, 0, 4] = k.astype(jnp.float32)
    out_ref[0, 0, 5] = 0.0
    out_ref[0, 0, 6] = 0.0
    out_ref[0, 0, 7] = 0.0


def kernel(loc_data, conf_data, obj_data, priors, targets):
    bsz = loc_data.shape[0]
    pad = PPAD - PP

    def prep(x):
        x = jnp.pad(x, ((0, 0), (0, pad), (0, 0)))
        return x.transpose(0, 2, 1).reshape(bsz, x.shape[2], RR, LL)

    loc_p = prep(loc_data)
    conf_p = prep(conf_data)
    obj_p = prep(obj_data)
    pri_p = jnp.pad(priors, ((0, pad), (0, 0))).T.reshape(4, RR, LL)
    tgt = targets.reshape(bsz, NT * 5)

    out = pl.pallas_call(
        _loss_body,
        grid=(bsz,),
        in_specs=[
            pl.BlockSpec(memory_space=pltpu.SMEM),
            pl.BlockSpec((4, RR, LL), lambda b: (0, 0, 0)),
            pl.BlockSpec((1, 4, RR, LL), lambda b: (b, 0, 0, 0)),
            pl.BlockSpec((1, NCLS, RR, LL), lambda b: (b, 0, 0, 0)),
            pl.BlockSpec((1, 2, RR, LL), lambda b: (b, 0, 0, 0)),
        ],
        out_specs=pl.BlockSpec((1, 1, NOUT), lambda b: (b, 0, 0),
                               memory_space=pltpu.SMEM),
        out_shape=jax.ShapeDtypeStruct((bsz, 1, NOUT), jnp.float32),
    )(tgt, pri_p, loc_p, conf_p, obj_p)

    o = out.reshape(bsz, NOUT)
    n_pos = jnp.maximum(jnp.sum(o[:, 3]), 1.0)
    n_neg = jnp.maximum(jnp.sum(o[:, 4]), 1.0)
    loss_l = jnp.sum(o[:, 0]) / n_pos
    loss_c = jnp.sum(o[:, 1]) / n_pos
    loss_obj = 0.4 * jnp.sum(o[:, 2]) / n_neg
    return (loss_l, loss_c, loss_obj)


# speedup vs baseline: 1.2276x; 1.0843x over previous
import jax
import jax.numpy as jnp
from jax.experimental import pallas as pl
from jax.experimental.pallas import tpu as pltpu

NCLS = 21
THR = 0.5
NEGPOS = 3
PP = 25500
RR = 200
LL = 128
PPAD = RR * LL
NT = 10
NOUT = 8


def _smooth_l1(d):
    ad = jnp.abs(d)
    return jnp.where(ad < 1.0, 0.5 * d * d, ad - 0.5)


def _loss_body(tgt_ref, pri_ref, loc_ref, conf_ref, obj_ref, out_ref):
    b = pl.program_id(0)

    pcx = pri_ref[0]
    pcy = pri_ref[1]
    pw = pri_ref[2]
    ph = pri_ref[3]
    px1 = pcx - pw * 0.5
    py1 = pcy - ph * 0.5
    px2 = pcx + pw * 0.5
    py2 = pcy + ph * 0.5
    parea = (px2 - px1) * (py2 - py1)

    row = jax.lax.broadcasted_iota(jnp.int32, (RR, LL), 0)
    col = jax.lax.broadcasted_iota(jnp.int32, (RR, LL), 1)
    p_idx = row * LL + col
    valid = p_idx < PP

    bto = jnp.full((RR, LL), -1.0, jnp.float32)
    bti = jnp.zeros((RR, LL), jnp.int32)
    tx1 = [tgt_ref[b, 5 * j + 0] for j in range(NT)]
    ty1 = [tgt_ref[b, 5 * j + 1] for j in range(NT)]
    tx2 = [tgt_ref[b, 5 * j + 2] for j in range(NT)]
    ty2 = [tgt_ref[b, 5 * j + 3] for j in range(NT)]
    tlab = [tgt_ref[b, 5 * j + 4] for j in range(NT)]
    bpi = []
    for j in range(NT):
        ix = jnp.maximum(jnp.minimum(px2, tx2[j]) - jnp.maximum(px1, tx1[j]), 0.0)
        iy = jnp.maximum(jnp.minimum(py2, ty2[j]) - jnp.maximum(py1, ty1[j]), 0.0)
        inter = ix * iy
        tarea = (tx2[j] - tx1[j]) * (ty2[j] - ty1[j])
        ov = inter / (tarea + parea - inter)
        ov = jnp.where(valid, ov, -1.0)
        upd = ov > bto
        bto = jnp.where(upd, ov, bto)
        bti = jnp.where(upd, j, bti)
        mj = jnp.max(ov)
        bpi.append(jnp.min(jnp.where(ov == mj, p_idx, PPAD)))
    for j in range(NT):
        hit = p_idx == bpi[j]
        bto = jnp.where(hit, 2.0, bto)
        bti = jnp.where(hit, j, bti)
    mx1 = jnp.zeros((RR, LL), jnp.float32)
    my1 = jnp.zeros((RR, LL), jnp.float32)
    mx2 = jnp.zeros((RR, LL), jnp.float32)
    my2 = jnp.zeros((RR, LL), jnp.float32)
    lab = jnp.zeros((RR, LL), jnp.float32)
    for j in range(NT):
        sel = bti == j
        mx1 = jnp.where(sel, tx1[j], mx1)
        my1 = jnp.where(sel, ty1[j], my1)
        mx2 = jnp.where(sel, tx2[j], mx2)
        my2 = jnp.where(sel, ty2[j], my2)
        lab = jnp.where(sel, tlab[j], lab)
    conf_t = jnp.where(bto < THR, 0, (lab + 1.0).astype(jnp.int32))
    pos = conf_t > 0

    g0 = ((mx1 + mx2) * 0.5 - pcx) / (0.1 * pw)
    g1 = ((my1 + my2) * 0.5 - pcy) / (0.1 * ph)
    g2 = jnp.log((mx2 - mx1) / pw) / 0.2
    g3 = jnp.log((my2 - my1) / ph) / 0.2
    sl1 = (_smooth_l1(loc_ref[0, 0] - g0) + _smooth_l1(loc_ref[0, 1] - g1)
           + _smooth_l1(loc_ref[0, 2] - g2) + _smooth_l1(loc_ref[0, 3] - g3))
    loss_l = jnp.sum(jnp.where(pos, sl1, 0.0))

    o0 = obj_ref[0, 0]
    o1 = obj_ref[0, 1]
    mo = jnp.maximum(o0, o1)
    lse_o = jnp.log(jnp.exp(o0 - mo) + jnp.exp(o1 - mo)) + mo
    ce_o = lse_o - jnp.where(pos, o1, o0)

    mc = conf_ref[0, 0]
    for i in range(1, NCLS):
        mc = jnp.maximum(mc, conf_ref[0, i])
    s = jnp.zeros((RR, LL), jnp.float32)
    chosen = jnp.zeros((RR, LL), jnp.float32)
    for i in range(NCLS):
        ci = conf_ref[0, i]
        s = s + jnp.exp(ci - mc)
        chosen = jnp.where(conf_t == i, ci, chosen)
    ce_c = (jnp.log(s) + mc) - chosen

    mine_c = jnp.where(valid, jnp.where(pos, 0.0, ce_c), -1.0)
    mine_o = jnp.where(valid, jnp.where(pos, 0.0, ce_o), -1.0)
    vi_c = jax.lax.bitcast_convert_type(mine_c, jnp.int32)
    vi_o = jax.lax.bitcast_convert_type(mine_o, jnp.int32)
    np_cnt = jnp.sum(pos.astype(jnp.int32))
    k = jnp.minimum(NEGPOS * np_cnt, PP - 1)

    def eight_way(lo, hi, cnt, vi):
        s = (hi - lo + 8) // 8
        idx = jnp.int32(0)
        new_cnt = cnt
        for i in range(1, 8):
            ci = jnp.sum((vi >= lo + i * s).astype(jnp.int32))
            ok = ci >= k
            idx = idx + ok.astype(jnp.int32)
            new_cnt = jnp.where(ok, ci, new_cnt)
        new_lo = lo + idx * s
        new_hi = jnp.minimum(hi, lo + (idx + 1) * s - 1)
        return new_lo, new_hi, new_cnt

    def _done(lo, hi, cnt):
        return jnp.logical_or(cnt == k, lo == hi)

    def bis_cond(st):
        lo_c, hi_c, cnt_c, lo_o, hi_o, cnt_o = st
        return jnp.logical_not(jnp.logical_and(_done(lo_c, hi_c, cnt_c),
                                               _done(lo_o, hi_o, cnt_o)))

    def bis(st):
        lo_c, hi_c, cnt_c, lo_o, hi_o, cnt_o = st
        nl, nh, nc = eight_way(lo_c, hi_c, cnt_c, vi_c)
        d = _done(lo_c, hi_c, cnt_c)
        lo_c = jnp.where(d, lo_c, nl)
        hi_c = jnp.where(d, hi_c, nh)
        cnt_c = jnp.where(d, cnt_c, nc)
        nl, nh, nc = eight_way(lo_o, hi_o, cnt_o, vi_o)
        d = _done(lo_o, hi_o, cnt_o)
        lo_o = jnp.where(d, lo_o, nl)
        hi_o = jnp.where(d, hi_o, nh)
        cnt_o = jnp.where(d, cnt_o, nc)
        return (lo_c, hi_c, cnt_c, lo_o, hi_o, cnt_o)

    big = jnp.int32(PPAD + 1)
    lo_c, _, cnt_c, lo_o, _, cnt_o = jax.lax.while_loop(
        bis_cond, bis,
        (jnp.int32(0), jnp.max(vi_c), big, jnp.int32(0), jnp.max(vi_o), big))

    def pick_t(lo, cnt, vi):
        mn = jnp.min(jnp.where(vi >= lo, vi, jnp.int32(2147483647)))
        return jnp.where(cnt == k, mn, lo)

    t_c = pick_t(lo_c, cnt_c, vi_c)
    t_o = pick_t(lo_o, cnt_o, vi_o)

    def neg_sum(mine, t_int):
        t = jax.lax.bitcast_convert_type(t_int, jnp.float32)
        gt = mine > t
        g_cnt = jnp.sum(gt.astype(jnp.int32))
        return (jnp.sum(jnp.where(gt, mine, 0.0))
                + (k - g_cnt).astype(jnp.float32) * t)

    loss_c = jnp.sum(jnp.where(pos, ce_c, 0.0)) + neg_sum(mine_c, t_c)
    loss_o = jnp.sum(jnp.where(pos, ce_o, 0.0)) + neg_sum(mine_o, t_o)

    out_ref[0, 0, 0] = loss_l
    out_ref[0, 0, 1] = loss_c
    out_ref[0, 0, 2] = loss_o
    out_ref[0, 0, 3] = np_cnt.astype(jnp.float32)
    out_ref[0, 0, 4] = k.astype(jnp.float32)
    out_ref[0, 0, 5] = 0.0
    out_ref[0, 0, 6] = 0.0
    out_ref[0, 0, 7] = 0.0


def kernel(loc_data, conf_data, obj_data, priors, targets):
    bsz = loc_data.shape[0]
    pad = PPAD - PP

    def prep(x):
        x = jnp.pad(x, ((0, 0), (0, pad), (0, 0)))
        return x.transpose(0, 2, 1).reshape(bsz, x.shape[2], RR, LL)

    loc_p = prep(loc_data)
    conf_p = prep(conf_data)
    obj_p = prep(obj_data)
    pri_p = jnp.pad(priors, ((0, pad), (0, 0))).T.reshape(4, RR, LL)
    tgt = targets.reshape(bsz, NT * 5)

    out = pl.pallas_call(
        _loss_body,
        grid=(bsz,),
        in_specs=[
            pl.BlockSpec(memory_space=pltpu.SMEM),
            pl.BlockSpec((4, RR, LL), lambda b: (0, 0, 0)),
            pl.BlockSpec((1, 4, RR, LL), lambda b: (b, 0, 0, 0)),
            pl.BlockSpec((1, NCLS, RR, LL), lambda b: (b, 0, 0, 0)),
            pl.BlockSpec((1, 2, RR, LL), lambda b: (b, 0, 0, 0)),
        ],
        out_specs=pl.BlockSpec((1, 1, NOUT), lambda b: (b, 0, 0),
                               memory_space=pltpu.SMEM),
        out_shape=jax.ShapeDtypeStruct((bsz, 1, NOUT), jnp.float32),
    )(tgt, pri_p, loc_p, conf_p, obj_p)

    o = out.reshape(bsz, NOUT)
    n_pos = jnp.maximum(jnp.sum(o[:, 3]), 1.0)
    n_neg = jnp.maximum(jnp.sum(o[:, 4]), 1.0)
    loss_l = jnp.sum(o[:, 0]) / n_pos
    loss_c = jnp.sum(o[:, 1]) / n_pos
    loss_obj = 0.4 * jnp.sum(o[:, 2]) / n_neg
    return (loss_l, loss_c, loss_obj)
